# baseline (speedup 1.0000x reference)
import functools

import jax
import jax.numpy as jnp
import numpy as np
from jax import lax
from jax.experimental import pallas as pl
from jax.experimental.pallas import tpu as pltpu
from jax.experimental.pallas import tpu_sc as plsc

H = 256
W = 256
N = 5000
NP = 5120
S = 10
K = 20
G5 = 51
NPIX = G5 * G5
PIX_TILE = 256
NTILES = 11
NPIX_PAD = NTILES * PIX_TILE
CH = 24
CHP = 32
NW = 32
GCHUNK = 128
NCHUNK = 13
B_PER_W = GCHUNK * NCHUNK
BTOT = B_PER_W * NW
ROWS_BLK = 8
DC = np.float32(256.0 / 50.0)
DF = np.float32(256.0 / 255.0)


def _knn_table_kernel(x_ref, nvec_ref, idcs_ref, table_ref):
    g = pl.program_id(0)
    p = g * PIX_TILE + lax.broadcasted_iota(jnp.int32, (PIX_TILE, 1), 0)
    pi = p // G5
    pj = p - pi * G5
    fi = pi.astype(jnp.float32) * DC
    fj = pj.astype(jnp.float32) * DC
    lx = x_ref[9:10, :]
    ly = x_ref[10:11, :]
    d = (fi - lx) ** 2 + (fj - ly) ** 2
    nvec = nvec_ref[...]
    cols = []
    big = jnp.float32(1e30)
    m = jnp.min(d, axis=1, keepdims=True)
    for k in range(K):
        eqf = (d == m).astype(jnp.float32)
        idxf = jnp.dot(eqf, nvec, preferred_element_type=jnp.float32)
        cols.append(jnp.minimum(idxf[:, 0:1].astype(jnp.int32),
                                jnp.int32(N - 1)))
        if k < K - 1:
            m = jnp.min(jnp.where(d > m, d, big), axis=1, keepdims=True)
    idcs_ref[...] = jnp.concatenate(cols, axis=1)

    @pl.when(g == 0)
    def _():
        t = (lax.broadcasted_iota(jnp.int32, (S, 1), 0).astype(jnp.float32)
             * jnp.float32(1.0 / 9.0))
        sx = x_ref[0:1, :] + lx
        sy = x_ref[1:2, :] + ly
        ex = x_ref[2:3, :] + lx
        ey = x_ref[3:4, :] + ly
        cx = x_ref[4:5, :] + lx
        cy = x_ref[5:6, :] + ly
        omt2 = (1.0 - t) ** 2
        t2 = t ** 2
        table_ref[0:S, :] = cx + omt2 * (sx - cx) + t2 * (ex - cx)
        table_ref[S:2 * S, :] = cy + omt2 * (sy - cy) + t2 * (ey - cy)
        table_ref[2 * S:2 * S + 3, :] = x_ref[6:9, :]
        table_ref[2 * S + 3:CH, :] = x_ref[11:12, :]


def _knn_and_table(x12, nvec):
    full = lambda shape: pl.BlockSpec(shape, lambda g: tuple(0 for _ in shape))
    return pl.pallas_call(
        _knn_table_kernel,
        grid=(NTILES,),
        in_specs=[full((128, NP)), full((NP, 128))],
        out_specs=[pl.BlockSpec((PIX_TILE, K), lambda g: (g, 0)),
                   full((CH, NP))],
        out_shape=[jax.ShapeDtypeStruct((NPIX_PAD, K), jnp.int32),
                   jax.ShapeDtypeStruct((CH, NP), jnp.float32)],
    )(x12, nvec)


def _gather_rows(table, idx_flat):
    mesh = plsc.VectorSubcoreMesh(core_axis_name="c", subcore_axis_name="s")

    @functools.partial(
        pl.kernel, mesh=mesh,
        compiler_params=pltpu.CompilerParams(use_tc_tiling_on_sc=False),
        out_type=jax.ShapeDtypeStruct((BTOT, CHP), jnp.float32),
        scratch_types=[
            pltpu.VMEM((B_PER_W,), jnp.int32),
            pltpu.VMEM((B_PER_W, CHP), jnp.float32),
            pltpu.SemaphoreType.DMA,
        ],
    )
    def k(table_hbm, idx_hbm, out_hbm, idx_v, rows_v, sem):
        wid = lax.axis_index("s") * 2 + lax.axis_index("c")
        base = wid * B_PER_W
        pltpu.sync_copy(idx_hbm.at[pl.ds(base, B_PER_W)], idx_v)
        copies = []
        for j in range(NCHUNK):
            copies.append(pltpu.async_copy(
                table_hbm.at[idx_v.at[pl.ds(j * GCHUNK, GCHUNK)]],
                rows_v.at[pl.ds(j * GCHUNK, GCHUNK), :], sem))
        for cp in copies:
            cp.wait()
        pltpu.sync_copy(rows_v, out_hbm.at[pl.ds(base, B_PER_W)])

    return k(table, idx_flat)


G5P = 56
CE_TILE = 2240


def _colexpand_kernel(gtm_ref, gwm_ref, cnt_ref, cbt_ref, gc_ref, gcw_ref):
    g = pl.program_id(0)
    gc_ref[...] = jnp.dot(gtm_ref[...], cnt_ref[...],
                          preferred_element_type=jnp.float32)

    @pl.when(g == 0)
    def _():
        gcw_ref[...] = jnp.dot(gwm_ref[...], cbt_ref[...],
                               preferred_element_type=jnp.float32)


def _colexpand(gtm, gwm, cntm, cbtm):
    full = lambda shape: pl.BlockSpec(shape, lambda g: tuple(0 for _ in shape))
    nrows = G5P * K * CH
    return pl.pallas_call(
        _colexpand_kernel,
        grid=(nrows // CE_TILE,),
        in_specs=[pl.BlockSpec((CE_TILE, G5), lambda g: (g, 0)),
                  full((G5P * K, G5)), full((G5, W)), full((G5, W))],
        out_specs=[pl.BlockSpec((CE_TILE, W), lambda g: (g, 0)),
                   full((G5P * K, W))],
        out_shape=[jax.ShapeDtypeStruct((nrows, W), jnp.float32),
                   jax.ShapeDtypeStruct((G5P * K, W), jnp.float32)],
    )(gtm, gwm, cntm, cbtm)


def _render_kernel(gc_ref, gcw_ref, rn_ref, rb_ref, o_ref):
    g = pl.program_id(0)
    i0 = 8 * g
    sn = jnp.minimum((jnp.int32(G5) * (2 * i0 + 1)) // (2 * H),
                     jnp.int32(G5 - 4))
    sb = jnp.clip(((2 * i0 + 1) * jnp.int32(G5) - H) // (2 * H),
                  jnp.int32(0), jnp.int32(G5 - 4))
    s8n = pl.multiple_of((sn // 8) * 8, 8)
    s8b = pl.multiple_of((sb // 8) * 8, 8)
    t2 = jnp.dot(rn_ref[...], gc_ref[pl.ds(s8n, 16), :],
                 preferred_element_type=jnp.float32)
    t2w = jnp.dot(rb_ref[...], gcw_ref[pl.ds(s8b, 16), :],
                  preferred_element_type=jnp.float32)
    px = ((g * ROWS_BLK
           + lax.broadcasted_iota(jnp.int32, (ROWS_BLK, 1), 0))
          .astype(jnp.float32) * DF)
    py = lax.broadcasted_iota(jnp.int32, (1, W), 1).astype(jnp.float32) * DF
    minds = []
    for k in range(K):
        o = k * CH * W
        mind_k = None
        for s in range(S - 1):
            ax = t2[:, o + s * W:o + (s + 1) * W]
            bx = t2[:, o + (s + 1) * W:o + (s + 2) * W]
            ay = t2[:, o + (S + s) * W:o + (S + s + 1) * W]
            by = t2[:, o + (S + s + 1) * W:o + (S + s + 2) * W]
            bax = bx - ax
            bay = by - ay
            tnum = bax * (px - ax) + bay * (py - ay)
            tden = bax * bax + bay * bay
            t = jnp.clip(tnum / tden, 0.0, 1.0)
            dx = px - (ax + t * bax)
            dy = py - (ay + t * bay)
            d = dx * dx + dy * dy
            mind_k = d if mind_k is None else jnp.minimum(mind_k, d)
        minds.append(mind_k)
    mind = jnp.stack(minds, axis=1)
    dmin = jnp.min(mind, axis=1)
    z = jnp.float32(100000.0) / (jnp.float32(1e-8) + mind)
    z = z - jnp.max(z, axis=1, keepdims=True)
    ez = jnp.exp(z)
    rank = ez / jnp.sum(ez, axis=1, keepdims=True)
    bs = jnp.zeros((ROWS_BLK, W), jnp.float32)
    i0 = jnp.zeros((ROWS_BLK, W), jnp.float32)
    i1 = jnp.zeros((ROWS_BLK, W), jnp.float32)
    i2 = jnp.zeros((ROWS_BLK, W), jnp.float32)
    for k in range(K):
        o = k * CH * W
        rk = rank[:, k, :]
        bs = bs + t2w[:, k * W:(k + 1) * W] * rk
        i0 = i0 + t2[:, o + 2 * S * W:o + (2 * S + 1) * W] * rk
        i1 = i1 + t2[:, o + (2 * S + 1) * W:o + (2 * S + 2) * W] * rk
        i2 = i2 + t2[:, o + (2 * S + 2) * W:o + (2 * S + 3) * W] * rk
    mask = 1.0 / (1.0 + jnp.exp(dmin - bs))
    cc = jnp.float32(0.5)
    o_ref[...] = jnp.stack([i0 * mask + (1.0 - mask) * cc,
                            i1 * mask + (1.0 - mask) * cc,
                            i2 * mask + (1.0 - mask) * cc], axis=0)


def _render(gc2, gcw2, rnm, rbm):
    full = lambda shape: pl.BlockSpec(shape, lambda g: tuple(0 for _ in shape))
    return pl.pallas_call(
        _render_kernel,
        grid=(H // ROWS_BLK,),
        in_specs=[full((G5P, K * CH * W)), full((G5P, K * W)),
                  pl.BlockSpec((ROWS_BLK, 16), lambda g: (g, 0)),
                  pl.BlockSpec((ROWS_BLK, 16), lambda g: (g, 0))],
        out_specs=pl.BlockSpec((3, ROWS_BLK, W), lambda g: (0, g, 0)),
        out_shape=jax.ShapeDtypeStruct((3, H, W), jnp.float32),
    )(gc2, gcw2, rnm, rbm)


def kernel(curve_s, curve_e, curve_c, color, location, width):
    x = jnp.concatenate([curve_s, curve_e, curve_c, color, location, width],
                        axis=1)
    x12 = jnp.pad(x, ((0, NP - N), (0, 128 - 12)), constant_values=1e9).T
    nvec = jnp.where(jnp.arange(128, dtype=jnp.int32)[None, :] == 0,
                     jnp.arange(NP, dtype=jnp.float32)[:, None],
                     jnp.float32(0.0))

    idcs, tableT = _knn_and_table(x12, nvec)

    table = jnp.pad(tableT.T, ((0, 0), (0, CHP - CH)))
    idx_flat = jnp.pad(idcs[:NPIX].reshape(-1), (0, BTOT - NPIX * K))
    gout = _gather_rows(table, idx_flat)

    g5 = gout[:NPIX * K].reshape(G5, G5, K, CHP)
    gtm = jnp.pad(g5[..., :CH].transpose(0, 2, 3, 1).reshape(G5 * K * CH, G5),
                  ((0, (G5P - G5) * K * CH), (0, 0)))
    gwm = jnp.pad(g5[..., CH - 1].transpose(0, 2, 1).reshape(G5 * K, G5),
                  ((0, (G5P - G5) * K), (0, 0)))

    eye = jnp.eye(G5, dtype=jnp.float32)
    rnm = jax.image.resize(eye, (H, G5), method='nearest')
    cntm = jax.image.resize(eye, (G5, W), method='nearest')
    rbm = jax.image.resize(eye, (H, G5), method='bilinear')
    cbtm = jax.image.resize(eye, (G5, W), method='bilinear')

    i = jnp.arange(H, dtype=jnp.int32)
    iblk = (i // ROWS_BLK) * ROWS_BLK
    sn_r = jnp.minimum((G5 * (2 * iblk + 1)) // (2 * H), G5 - 4)
    sb_r = jnp.clip(((2 * iblk + 1) * G5 - H) // (2 * H), 0, G5 - 4)
    s8n_r = (sn_r // 8) * 8
    s8b_r = (sb_r // 8) * 8
    cols16 = jnp.arange(16, dtype=jnp.int32)[None, :]
    rnm_p = jnp.pad(rnm, ((0, 0), (0, G5P - G5)))
    rbm_p = jnp.pad(rbm, ((0, 0), (0, G5P - G5)))
    rnw = jnp.take_along_axis(rnm_p, s8n_r[:, None] + cols16, axis=1)
    rbw = jnp.take_along_axis(rbm_p, s8b_r[:, None] + cols16, axis=1)

    gc, gcw = _colexpand(gtm, gwm, cntm, cbtm)
    out = _render(gc.reshape(G5P, K * CH * W), gcw.reshape(G5P, K * W),
                  rnw, rbw)
    return jnp.transpose(out, (1, 2, 0))


# speedup vs baseline: 1.0368x; 1.0368x over previous
import functools

import jax
import jax.numpy as jnp
import numpy as np
from jax import lax
from jax.experimental import pallas as pl
from jax.experimental.pallas import tpu as pltpu
from jax.experimental.pallas import tpu_sc as plsc

H = 256
W = 256
N = 5000
NP = 5120
S = 10
K = 20
G5 = 51
NPIX = G5 * G5
PIX_TILE = 128
NTILES = 21
NPIX_PAD = NTILES * PIX_TILE
CH = 24
CHP = 32
NW = 32
GCHUNK = 128
NCHUNK = 13
B_PER_W = GCHUNK * NCHUNK
BTOT = B_PER_W * NW
ROWS_BLK = 8
DC = np.float32(256.0 / 50.0)
DF = np.float32(256.0 / 255.0)


def _knn_table_kernel(x_ref, nvec_ref, idcs_ref, table_ref):
    g = pl.program_id(0)
    p = g * PIX_TILE + lax.broadcasted_iota(jnp.int32, (PIX_TILE, 1), 0)
    pi = p // G5
    pj = p - pi * G5
    fi = pi.astype(jnp.float32) * DC
    fj = pj.astype(jnp.float32) * DC
    lx = x_ref[9:10, :]
    ly = x_ref[10:11, :]
    d = (fi - lx) ** 2 + (fj - ly) ** 2
    nvec = nvec_ref[...]
    cols = []
    big = jnp.float32(1e30)
    m = jnp.min(d, axis=1, keepdims=True)
    tprev = jnp.full((PIX_TILE, 1), jnp.float32(NP * (NP - 1) // 2))
    for k in range(K):
        gtb = d > m
        gtf = gtb.astype(jnp.float32)
        tk = jnp.dot(gtf, nvec, preferred_element_type=jnp.float32)[:, 0:1]
        cols.append(jnp.minimum((tprev - tk).astype(jnp.int32),
                                jnp.int32(N - 1)))
        tprev = tk
        if k < K - 1:
            m = jnp.min(jnp.where(gtb, d, big), axis=1, keepdims=True)
    idcs_ref[...] = jnp.concatenate(cols, axis=1)

    @pl.when(g == 0)
    def _():
        t = (lax.broadcasted_iota(jnp.int32, (S, 1), 0).astype(jnp.float32)
             * jnp.float32(1.0 / 9.0))
        sx = x_ref[0:1, :] + lx
        sy = x_ref[1:2, :] + ly
        ex = x_ref[2:3, :] + lx
        ey = x_ref[3:4, :] + ly
        cx = x_ref[4:5, :] + lx
        cy = x_ref[5:6, :] + ly
        omt2 = (1.0 - t) ** 2
        t2 = t ** 2
        table_ref[0:S, :] = cx + omt2 * (sx - cx) + t2 * (ex - cx)
        table_ref[S:2 * S, :] = cy + omt2 * (sy - cy) + t2 * (ey - cy)
        table_ref[2 * S:2 * S + 3, :] = x_ref[6:9, :]
        table_ref[2 * S + 3:CH, :] = x_ref[11:12, :]


def _knn_and_table(x12, nvec):
    full = lambda shape: pl.BlockSpec(shape, lambda g: tuple(0 for _ in shape))
    return pl.pallas_call(
        _knn_table_kernel,
        grid=(NTILES,),
        in_specs=[full((128, NP)), full((NP, 128))],
        out_specs=[pl.BlockSpec((PIX_TILE, K), lambda g: (g, 0)),
                   full((CH, NP))],
        out_shape=[jax.ShapeDtypeStruct((NPIX_PAD, K), jnp.int32),
                   jax.ShapeDtypeStruct((CH, NP), jnp.float32)],
    )(x12, nvec)


def _gather_rows(table, idx_flat):
    mesh = plsc.VectorSubcoreMesh(core_axis_name="c", subcore_axis_name="s")

    @functools.partial(
        pl.kernel, mesh=mesh,
        compiler_params=pltpu.CompilerParams(use_tc_tiling_on_sc=False),
        out_type=jax.ShapeDtypeStruct((BTOT, CHP), jnp.float32),
        scratch_types=[
            pltpu.VMEM((B_PER_W,), jnp.int32),
            pltpu.VMEM((B_PER_W, CHP), jnp.float32),
            pltpu.SemaphoreType.DMA,
        ],
    )
    def k(table_hbm, idx_hbm, out_hbm, idx_v, rows_v, sem):
        wid = lax.axis_index("s") * 2 + lax.axis_index("c")
        base = wid * B_PER_W
        pltpu.sync_copy(idx_hbm.at[pl.ds(base, B_PER_W)], idx_v)
        copies = []
        for j in range(NCHUNK):
            copies.append(pltpu.async_copy(
                table_hbm.at[idx_v.at[pl.ds(j * GCHUNK, GCHUNK)]],
                rows_v.at[pl.ds(j * GCHUNK, GCHUNK), :], sem))
        for cp in copies:
            cp.wait()
        pltpu.sync_copy(rows_v, out_hbm.at[pl.ds(base, B_PER_W)])

    return k(table, idx_flat)


G5P = 56
CE_TILE = 2240


def _colexpand_kernel(gtm_ref, gwm_ref, cnt_ref, cbt_ref, gc_ref, gcw_ref):
    g = pl.program_id(0)
    gc_ref[...] = jnp.dot(gtm_ref[...], cnt_ref[...],
                          preferred_element_type=jnp.float32)

    @pl.when(g == 0)
    def _():
        gcw_ref[...] = jnp.dot(gwm_ref[...], cbt_ref[...],
                               preferred_element_type=jnp.float32)


def _colexpand(gtm, gwm, cntm, cbtm):
    full = lambda shape: pl.BlockSpec(shape, lambda g: tuple(0 for _ in shape))
    nrows = G5P * K * CH
    return pl.pallas_call(
        _colexpand_kernel,
        grid=(nrows // CE_TILE,),
        in_specs=[pl.BlockSpec((CE_TILE, G5), lambda g: (g, 0)),
                  full((G5P * K, G5)), full((G5, W)), full((G5, W))],
        out_specs=[pl.BlockSpec((CE_TILE, W), lambda g: (g, 0)),
                   full((G5P * K, W))],
        out_shape=[jax.ShapeDtypeStruct((nrows, W), jnp.float32),
                   jax.ShapeDtypeStruct((G5P * K, W), jnp.float32)],
    )(gtm, gwm, cntm, cbtm)


def _render_kernel(gc_ref, gcw_ref, rn_ref, rb_ref, o_ref):
    g = pl.program_id(0)
    i0 = 8 * g
    sn = jnp.minimum((jnp.int32(G5) * (2 * i0 + 1)) // (2 * H),
                     jnp.int32(G5 - 4))
    sb = jnp.clip(((2 * i0 + 1) * jnp.int32(G5) - H) // (2 * H),
                  jnp.int32(0), jnp.int32(G5 - 4))
    s8n = pl.multiple_of((sn // 8) * 8, 8)
    s8b = pl.multiple_of((sb // 8) * 8, 8)
    t2 = jnp.dot(rn_ref[...], gc_ref[pl.ds(s8n, 16), :],
                 preferred_element_type=jnp.float32)
    t2w = jnp.dot(rb_ref[...], gcw_ref[pl.ds(s8b, 16), :],
                  preferred_element_type=jnp.float32)
    px = ((g * ROWS_BLK
           + lax.broadcasted_iota(jnp.int32, (ROWS_BLK, 1), 0))
          .astype(jnp.float32) * DF)
    py = lax.broadcasted_iota(jnp.int32, (1, W), 1).astype(jnp.float32) * DF
    minds = []
    for k in range(K):
        o = k * CH * W
        mind_k = None
        for s in range(S - 1):
            ax = t2[:, o + s * W:o + (s + 1) * W]
            bx = t2[:, o + (s + 1) * W:o + (s + 2) * W]
            ay = t2[:, o + (S + s) * W:o + (S + s + 1) * W]
            by = t2[:, o + (S + s + 1) * W:o + (S + s + 2) * W]
            bax = bx - ax
            bay = by - ay
            tnum = bax * (px - ax) + bay * (py - ay)
            tden = bax * bax + bay * bay
            t = jnp.clip(tnum / tden, 0.0, 1.0)
            dx = px - (ax + t * bax)
            dy = py - (ay + t * bay)
            d = dx * dx + dy * dy
            mind_k = d if mind_k is None else jnp.minimum(mind_k, d)
        minds.append(mind_k)
    mind = jnp.stack(minds, axis=1)
    dmin = jnp.min(mind, axis=1)
    z = jnp.float32(100000.0) / (jnp.float32(1e-8) + mind)
    z = z - jnp.max(z, axis=1, keepdims=True)
    ez = jnp.exp(z)
    rank = ez / jnp.sum(ez, axis=1, keepdims=True)
    bs = jnp.zeros((ROWS_BLK, W), jnp.float32)
    i0 = jnp.zeros((ROWS_BLK, W), jnp.float32)
    i1 = jnp.zeros((ROWS_BLK, W), jnp.float32)
    i2 = jnp.zeros((ROWS_BLK, W), jnp.float32)
    for k in range(K):
        o = k * CH * W
        rk = rank[:, k, :]
        bs = bs + t2w[:, k * W:(k + 1) * W] * rk
        i0 = i0 + t2[:, o + 2 * S * W:o + (2 * S + 1) * W] * rk
        i1 = i1 + t2[:, o + (2 * S + 1) * W:o + (2 * S + 2) * W] * rk
        i2 = i2 + t2[:, o + (2 * S + 2) * W:o + (2 * S + 3) * W] * rk
    mask = 1.0 / (1.0 + jnp.exp(dmin - bs))
    cc = jnp.float32(0.5)
    o_ref[...] = jnp.stack([i0 * mask + (1.0 - mask) * cc,
                            i1 * mask + (1.0 - mask) * cc,
                            i2 * mask + (1.0 - mask) * cc], axis=0)


def _render(gc2, gcw2, rnm, rbm):
    full = lambda shape: pl.BlockSpec(shape, lambda g: tuple(0 for _ in shape))
    return pl.pallas_call(
        _render_kernel,
        grid=(H // ROWS_BLK,),
        in_specs=[full((G5P, K * CH * W)), full((G5P, K * W)),
                  pl.BlockSpec((ROWS_BLK, 16), lambda g: (g, 0)),
                  pl.BlockSpec((ROWS_BLK, 16), lambda g: (g, 0))],
        out_specs=pl.BlockSpec((3, ROWS_BLK, W), lambda g: (0, g, 0)),
        out_shape=jax.ShapeDtypeStruct((3, H, W), jnp.float32),
    )(gc2, gcw2, rnm, rbm)


def kernel(curve_s, curve_e, curve_c, color, location, width):
    x = jnp.concatenate([curve_s, curve_e, curve_c, color, location, width],
                        axis=1)
    x12 = jnp.pad(x, ((0, NP - N), (0, 128 - 12)), constant_values=1e9).T
    nvec = jnp.where(jnp.arange(128, dtype=jnp.int32)[None, :] == 0,
                     jnp.arange(NP, dtype=jnp.float32)[:, None],
                     jnp.float32(0.0))

    idcs, tableT = _knn_and_table(x12, nvec)

    table = jnp.pad(tableT.T, ((0, 0), (0, CHP - CH)))
    idx_flat = jnp.pad(idcs[:NPIX].reshape(-1), (0, BTOT - NPIX * K))
    gout = _gather_rows(table, idx_flat)

    g5 = gout[:NPIX * K].reshape(G5, G5, K, CHP)
    gtm = jnp.pad(g5[..., :CH].transpose(0, 2, 3, 1).reshape(G5 * K * CH, G5),
                  ((0, (G5P - G5) * K * CH), (0, 0)))
    gwm = jnp.pad(g5[..., CH - 1].transpose(0, 2, 1).reshape(G5 * K, G5),
                  ((0, (G5P - G5) * K), (0, 0)))

    eye = jnp.eye(G5, dtype=jnp.float32)
    rnm = jax.image.resize(eye, (H, G5), method='nearest')
    cntm = jax.image.resize(eye, (G5, W), method='nearest')
    rbm = jax.image.resize(eye, (H, G5), method='bilinear')
    cbtm = jax.image.resize(eye, (G5, W), method='bilinear')

    i = jnp.arange(H, dtype=jnp.int32)
    iblk = (i // ROWS_BLK) * ROWS_BLK
    sn_r = jnp.minimum((G5 * (2 * iblk + 1)) // (2 * H), G5 - 4)
    sb_r = jnp.clip(((2 * iblk + 1) * G5 - H) // (2 * H), 0, G5 - 4)
    s8n_r = (sn_r // 8) * 8
    s8b_r = (sb_r // 8) * 8
    cols16 = jnp.arange(16, dtype=jnp.int32)[None, :]
    rnm_p = jnp.pad(rnm, ((0, 0), (0, G5P - G5)))
    rbm_p = jnp.pad(rbm, ((0, 0), (0, G5P - G5)))
    rnw = jnp.take_along_axis(rnm_p, s8n_r[:, None] + cols16, axis=1)
    rbw = jnp.take_along_axis(rbm_p, s8b_r[:, None] + cols16, axis=1)

    gc, gcw = _colexpand(gtm, gwm, cntm, cbtm)
    out = _render(gc.reshape(G5P, K * CH * W), gcw.reshape(G5P, K * W),
                  rnw, rbw)
    return jnp.transpose(out, (1, 2, 0))
